# baseline (speedup 1.0000x reference)
import functools

import jax
import jax.numpy as jnp
from jax import lax
from jax.experimental import pallas as pl
from jax.experimental.pallas import tpu as pltpu
from jax.experimental.pallas import tpu_sc as plsc

N = 10000
E = 320000
D = 128
EPS = 1e-5

NC = 2
NS = 16
NW = NC * NS
EW = E // NW
CH = 80
NP = 10240
NROWS_W = NP // NS
ZCH = 128

BN_BLK = 1000
BE_BLK = 1280
PARTS = (25, 75, 25)

_sc_mesh = plsc.VectorSubcoreMesh(core_axis_name="c", subcore_axis_name="s")


def _pre_body(x_ref, wu_ref, wv_ref, c1_ref, u_ref, v_ref):
    x = x_ref[...]
    u_ref[...] = jnp.dot(x, wu_ref[...], preferred_element_type=jnp.float32) + c1_ref[...]
    v_ref[...] = jnp.dot(x, wv_ref[...], preferred_element_type=jnp.float32)


def _edge_body(z_ref, w2_ref, b2_ref, w3_ref, b3_ref, m_ref):
    h1 = jnp.maximum(z_ref[...], 0.0)
    h2 = jnp.dot(h1, w2_ref[...], preferred_element_type=jnp.float32) + b2_ref[...]
    h2 = jnp.maximum(h2, 0.0)
    m_ref[...] = jnp.dot(h2, w3_ref[...], preferred_element_type=jnp.float32) + b3_ref[...]


def _node_body(x_ref, *refs):
    nparts = len(PARTS)
    ps = refs[:nparts]
    w1a_ref, w1b_ref, c1_ref, w2_ref, b2_ref, w3_ref, b3_ref, o_ref = refs[nparts:]
    x = x_ref[...]
    a = ps[0][0] + ps[0][1]
    for p_ref in ps[1:]:
        a = a + p_ref[0] + p_ref[1]
    g = jnp.dot(x, w1a_ref[...], preferred_element_type=jnp.float32)
    g = g + jnp.dot(a, w1b_ref[...], preferred_element_type=jnp.float32)
    g = jnp.maximum(g + c1_ref[...], 0.0)
    g = jnp.dot(g, w2_ref[...], preferred_element_type=jnp.float32) + b2_ref[...]
    g = jnp.maximum(g, 0.0)
    o_ref[...] = jnp.dot(g, w3_ref[...], preferred_element_type=jnp.float32) + b3_ref[...]


_W_SPEC = pl.BlockSpec((D, D), lambda i: (0, 0))
_B_SPEC = pl.BlockSpec((1, D), lambda i: (0, 0))


BP_BLK = 2000


def _run_pre(x, wu, wv, c1):
    return pl.pallas_call(
        _pre_body,
        grid=(N // BP_BLK,),
        in_specs=[pl.BlockSpec((BP_BLK, D), lambda i: (i, 0)), _W_SPEC, _W_SPEC, _B_SPEC],
        out_specs=(pl.BlockSpec((BP_BLK, D), lambda i: (i, 0)),
                   pl.BlockSpec((BP_BLK, D), lambda i: (i, 0))),
        out_shape=(jax.ShapeDtypeStruct((N, D), jnp.float32),
                   jax.ShapeDtypeStruct((N, D), jnp.float32)),
    )(x, wu, wv, c1)


def _run_edge_mlp(z, w2, b2, w3, b3, e_part):
    return pl.pallas_call(
        _edge_body,
        grid=(e_part // BE_BLK,),
        in_specs=[pl.BlockSpec((BE_BLK, D), lambda i: (i, 0)), _W_SPEC, _B_SPEC, _W_SPEC, _B_SPEC],
        out_specs=pl.BlockSpec((BE_BLK, D), lambda i: (i, 0)),
        out_shape=jax.ShapeDtypeStruct((e_part, D), jnp.float32),
    )(z, w2, b2, w3, b3)


def _run_node_mlp(x, partial_list, w1a, w1b, c1, w2, b2, w3, b3):
    p_spec = pl.BlockSpec((NC, BN_BLK, D), lambda i: (0, i, 0))
    return pl.pallas_call(
        _node_body,
        grid=(N // BN_BLK,),
        in_specs=[pl.BlockSpec((BN_BLK, D), lambda i: (i, 0))]
                 + [p_spec] * len(partial_list)
                 + [_W_SPEC, _W_SPEC, _B_SPEC, _W_SPEC, _B_SPEC, _W_SPEC, _B_SPEC],
        out_specs=pl.BlockSpec((BN_BLK, D), lambda i: (i, 0)),
        out_shape=jax.ShapeDtypeStruct((N, D), jnp.float32),
    )(x, *partial_list, w1a, w1b, c1, w2, b2, w3, b3)


NCHUNK = EW // CH


def _ring3(nchunk, process, prefetch, wait_drain):
    prefetch(0, 0)
    prefetch(1, 1)
    process(0, 0)
    prefetch(2, 2)
    process(1, 1)
    wait_drain(0)
    prefetch(3, 0)

    def full(k, b):
        process(k, b)
        wait_drain((b + 2) % 3)
        prefetch(k + 2, (b + 2) % 3)

    body_count = nchunk - 4
    triples = body_count // 3
    rem = body_count % 3

    if triples > 0:
        def triple(g, _):
            k0 = 2 + 3 * g
            full(k0, 2)
            full(k0 + 1, 0)
            full(k0 + 2, 1)
            return 0

        lax.fori_loop(0, triples, triple, 0)
    for j in range(rem):
        k = 2 + 3 * triples + j
        full(k, k % 3)
    process(nchunk - 2, (nchunk - 2) % 3)
    process(nchunk - 1, (nchunk - 1) % 3)
    wait_drain(0)
    wait_drain(1)
    wait_drain(2)


def _make_gather_body(nchunk):
    ew = nchunk * CH

    def body(u_hbm, v_hbm, dst_hbm, src_hbm, z_hbm,
             idx_d, idx_s, bu0, bv0, bu1, bv1, bu2, bv2,
             gs0, gs1, gs2, ws0, ws1, ws2):
        bus, bvs = (bu0, bu1, bu2), (bv0, bv1, bv2)
        gss, wss = (gs0, gs1, gs2), (ws0, ws1, ws2)
        c = lax.axis_index("c")
        s = lax.axis_index("s")
        wid = s * NC + c
        base0 = wid * ew
        cp_d = pltpu.async_copy(dst_hbm.at[pl.ds(base0, ew)], idx_d, gss[0])
        cp_s = pltpu.async_copy(src_hbm.at[pl.ds(base0, ew)], idx_s, gss[1])
        cp_d.wait()
        cp_s.wait()

        def prefetch(k, b):
            off = pl.ds(k * CH, CH)
            pltpu.async_copy(u_hbm.at[idx_d.at[off]], bus[b], gss[b])
            pltpu.async_copy(v_hbm.at[idx_s.at[off]], bvs[b], gss[b])

        def process(k, b):
            pltpu.make_async_copy(u_hbm.at[pl.ds(0, CH)], bus[b], gss[b]).wait()
            pltpu.make_async_copy(v_hbm.at[pl.ds(0, CH)], bvs[b], gss[b]).wait()

            def row(r, _):
                for j in range(D // 16):
                    sl = pl.ds(j * 16, 16)
                    plsc.addupdate(bus[b].at[r, sl], bvs[b][r, sl])
                return 0

            lax.fori_loop(0, CH, row, 0, unroll=4)
            pltpu.async_copy(bus[b], z_hbm.at[pl.ds(base0 + k * CH, CH)], wss[b])

        def wait_drain(b):
            pltpu.make_async_copy(z_hbm.at[pl.ds(0, CH)], bus[b], wss[b]).wait()

        _ring3(nchunk, process, prefetch, wait_drain)

    return body


def _run_gather(u, v, dst, src, nchunk):
    e_part = nchunk * CH * NW
    return pl.kernel(
        _make_gather_body(nchunk),
        mesh=_sc_mesh,
        out_type=jax.ShapeDtypeStruct((e_part, D), jnp.float32),
        scratch_types=[
            pltpu.VMEM((nchunk * CH,), jnp.int32),
            pltpu.VMEM((nchunk * CH,), jnp.int32),
            pltpu.VMEM((CH, D), jnp.float32),
            pltpu.VMEM((CH, D), jnp.float32),
            pltpu.VMEM((CH, D), jnp.float32),
            pltpu.VMEM((CH, D), jnp.float32),
            pltpu.VMEM((CH, D), jnp.float32),
            pltpu.VMEM((CH, D), jnp.float32),
            pltpu.SemaphoreType.DMA,
            pltpu.SemaphoreType.DMA,
            pltpu.SemaphoreType.DMA,
            pltpu.SemaphoreType.DMA,
            pltpu.SemaphoreType.DMA,
            pltpu.SemaphoreType.DMA,
        ],
    )(u, v, dst, src)


def _make_scatter_body(nchunk):
    ew = nchunk * CH

    def body(msg_hbm, dst3d_hbm, out_hbm, idx2d, m0, m1, m2, accum,
             isem, l0, l1, l2, s0, s1, s2):
        ms = (m0, m1, m2)
        lss, sss = (l0, l1, l2), (s0, s1, s2)
        c = lax.axis_index("c")
        s = lax.axis_index("s")
        wid = c * NS + s
        base0 = wid * ew

        cp_idx = pltpu.async_copy(dst3d_hbm.at[wid], idx2d, isem)

        def zrow(r, _):
            for j in range(D // 16):
                m0[r, pl.ds(j * 16, 16)] = jnp.zeros((16,), jnp.float32)
            return 0

        lax.fori_loop(0, CH, zrow, 0)
        for t in range(NROWS_W // CH):
            pltpu.sync_copy(m0, accum.at[pl.ds(s * NROWS_W + t * CH, CH)])
        cp_idx.wait()
        plsc.subcore_barrier()

        def prefetch(k, b):
            pltpu.async_copy(msg_hbm.at[pl.ds(base0 + k * CH, CH)], ms[b], lss[b])

        def process(k, b):
            pltpu.make_async_copy(msg_hbm.at[pl.ds(0, CH)], ms[b], lss[b]).wait()
            pltpu.async_copy(ms[b], accum.at[idx2d.at[k]], sss[b], add=True)

        def wait_drain(b):
            pltpu.make_async_copy(msg_hbm.at[pl.ds(0, CH)], ms[b], sss[b]).wait()

        _ring3(nchunk, process, prefetch, wait_drain)
        plsc.subcore_barrier()

        for t in range(NROWS_W // CH):
            sl = pl.ds(s * NROWS_W + t * CH, CH)
            pltpu.sync_copy(accum.at[sl], out_hbm.at[c, sl])

    return body


def _run_scatter(msg, dst3d, nchunk):
    return pl.kernel(
        _make_scatter_body(nchunk),
        mesh=_sc_mesh,
        out_type=jax.ShapeDtypeStruct((NC, NP, D), jnp.float32),
        scratch_types=[
            pltpu.VMEM((nchunk, CH), jnp.int32),
            pltpu.VMEM((CH, D), jnp.float32),
            pltpu.VMEM((CH, D), jnp.float32),
            pltpu.VMEM((CH, D), jnp.float32),
            pltpu.VMEM_SHARED((NP, D), jnp.float32),
            pltpu.SemaphoreType.DMA,
            pltpu.SemaphoreType.DMA,
            pltpu.SemaphoreType.DMA,
            pltpu.SemaphoreType.DMA,
            pltpu.SemaphoreType.DMA,
            pltpu.SemaphoreType.DMA,
            pltpu.SemaphoreType.DMA,
        ],
    )(msg, dst3d)


def kernel(pos, feat, edge_index, mW1, mb1, mg1, mbe1, mW2, mb2, mg2, mbe2,
           mW3, mb3, uW1, ub1, ug1, ube1, uW2, ub2, ug2, ube2, uW3, ub3):
    src = edge_index[0]
    dst = edge_index[1]
    x = jnp.concatenate([pos, feat], axis=1)

    s1 = mg1 / jnp.sqrt(1.0 + EPS)
    wp, wfi, wfj = mW1[0:2], mW1[2:D], mW1[D:2 * D - 2]
    wu = jnp.concatenate([-wp, wfi], axis=0) * s1
    wv = jnp.concatenate([wp, wfj], axis=0) * s1
    c1 = (mb1 * s1 + mbe1).reshape(1, D)

    s2 = mg2 / jnp.sqrt(1.0 + EPS)
    w2 = mW2 * s2
    b2 = (mb2 * s2 + mbe2).reshape(1, D)
    b3 = mb3.reshape(1, D)

    su1 = ug1 / jnp.sqrt(1.0 + EPS)
    w1a = uW1[:D] * su1
    w1b = uW1[D:] * su1
    cu1 = (ub1 * su1 + ube1).reshape(1, D)
    su2 = ug2 / jnp.sqrt(1.0 + EPS)
    uw2 = uW2 * su2
    bu2 = (ub2 * su2 + ube2).reshape(1, D)
    bu3 = ub3.reshape(1, D)

    u, v = _run_pre(x, wu, wv, c1)
    partial_list = []
    off = 0
    for units in PARTS:
        e_part = units * CH * NW
        dstp = lax.slice_in_dim(dst, off, off + e_part)
        srcp = lax.slice_in_dim(src, off, off + e_part)
        z = _run_gather(u, v, dstp, srcp, units)
        m = _run_edge_mlp(z, w2, b2, mW3, b3, e_part)
        partial_list.append(_run_scatter(m, dstp.reshape(NW, units, CH), units))
        off += e_part
    return _run_node_mlp(x, partial_list, w1a, w1b, cu1, uw2, bu2, uW3, bu3)


# speedup vs baseline: 1.0473x; 1.0473x over previous
import functools

import jax
import jax.numpy as jnp
from jax import lax
from jax.experimental import pallas as pl
from jax.experimental.pallas import tpu as pltpu
from jax.experimental.pallas import tpu_sc as plsc

N = 10000
E = 320000
D = 128
EPS = 1e-5

NC = 2
NS = 16
NW = NC * NS
EW = E // NW
CH = 80
NP = 10240
NROWS_W = NP // NS
ZCH = 128

BN_BLK = 1000
BE_BLK = 1280
PARTS = (20, 35, 35, 35)

_sc_mesh = plsc.VectorSubcoreMesh(core_axis_name="c", subcore_axis_name="s")


def _pre_body(x_ref, wu_ref, wv_ref, c1_ref, u_ref, v_ref):
    x = x_ref[...]
    u_ref[...] = jnp.dot(x, wu_ref[...], preferred_element_type=jnp.float32) + c1_ref[...]
    v_ref[...] = jnp.dot(x, wv_ref[...], preferred_element_type=jnp.float32)


def _edge_body(z_ref, w2_ref, b2_ref, w3_ref, b3_ref, m_ref):
    h1 = jnp.maximum(z_ref[...], 0.0)
    h2 = jnp.dot(h1, w2_ref[...], preferred_element_type=jnp.float32) + b2_ref[...]
    h2 = jnp.maximum(h2, 0.0)
    m_ref[...] = jnp.dot(h2, w3_ref[...], preferred_element_type=jnp.float32) + b3_ref[...]


def _node_body(x_ref, *refs):
    nparts = len(PARTS)
    ps = refs[:nparts]
    w1a_ref, w1b_ref, c1_ref, w2_ref, b2_ref, w3_ref, b3_ref, o_ref = refs[nparts:]
    x = x_ref[...]
    a = ps[0][0] + ps[0][1]
    for p_ref in ps[1:]:
        a = a + p_ref[0] + p_ref[1]
    g = jnp.dot(x, w1a_ref[...], preferred_element_type=jnp.float32)
    g = g + jnp.dot(a, w1b_ref[...], preferred_element_type=jnp.float32)
    g = jnp.maximum(g + c1_ref[...], 0.0)
    g = jnp.dot(g, w2_ref[...], preferred_element_type=jnp.float32) + b2_ref[...]
    g = jnp.maximum(g, 0.0)
    o_ref[...] = jnp.dot(g, w3_ref[...], preferred_element_type=jnp.float32) + b3_ref[...]


_W_SPEC = pl.BlockSpec((D, D), lambda i: (0, 0))
_B_SPEC = pl.BlockSpec((1, D), lambda i: (0, 0))


BP_BLK = 2000


def _run_pre(x, wu, wv, c1):
    return pl.pallas_call(
        _pre_body,
        grid=(N // BP_BLK,),
        in_specs=[pl.BlockSpec((BP_BLK, D), lambda i: (i, 0)), _W_SPEC, _W_SPEC, _B_SPEC],
        out_specs=(pl.BlockSpec((BP_BLK, D), lambda i: (i, 0)),
                   pl.BlockSpec((BP_BLK, D), lambda i: (i, 0))),
        out_shape=(jax.ShapeDtypeStruct((N, D), jnp.float32),
                   jax.ShapeDtypeStruct((N, D), jnp.float32)),
    )(x, wu, wv, c1)


def _run_edge_mlp(z, w2, b2, w3, b3, e_part):
    return pl.pallas_call(
        _edge_body,
        grid=(e_part // BE_BLK,),
        in_specs=[pl.BlockSpec((BE_BLK, D), lambda i: (i, 0)), _W_SPEC, _B_SPEC, _W_SPEC, _B_SPEC],
        out_specs=pl.BlockSpec((BE_BLK, D), lambda i: (i, 0)),
        out_shape=jax.ShapeDtypeStruct((e_part, D), jnp.float32),
    )(z, w2, b2, w3, b3)


def _run_node_mlp(x, partial_list, w1a, w1b, c1, w2, b2, w3, b3):
    p_spec = pl.BlockSpec((NC, BN_BLK, D), lambda i: (0, i, 0))
    return pl.pallas_call(
        _node_body,
        grid=(N // BN_BLK,),
        in_specs=[pl.BlockSpec((BN_BLK, D), lambda i: (i, 0))]
                 + [p_spec] * len(partial_list)
                 + [_W_SPEC, _W_SPEC, _B_SPEC, _W_SPEC, _B_SPEC, _W_SPEC, _B_SPEC],
        out_specs=pl.BlockSpec((BN_BLK, D), lambda i: (i, 0)),
        out_shape=jax.ShapeDtypeStruct((N, D), jnp.float32),
    )(x, *partial_list, w1a, w1b, c1, w2, b2, w3, b3)


NCHUNK = EW // CH


def _ring3(nchunk, process, prefetch, wait_drain):
    prefetch(0, 0)
    prefetch(1, 1)
    process(0, 0)
    prefetch(2, 2)
    process(1, 1)
    wait_drain(0)
    prefetch(3, 0)

    def full(k, b):
        process(k, b)
        wait_drain((b + 2) % 3)
        prefetch(k + 2, (b + 2) % 3)

    body_count = nchunk - 4
    triples = body_count // 3
    rem = body_count % 3

    if triples > 0:
        def triple(g, _):
            k0 = 2 + 3 * g
            full(k0, 2)
            full(k0 + 1, 0)
            full(k0 + 2, 1)
            return 0

        lax.fori_loop(0, triples, triple, 0)
    for j in range(rem):
        k = 2 + 3 * triples + j
        full(k, k % 3)
    process(nchunk - 2, (nchunk - 2) % 3)
    process(nchunk - 1, (nchunk - 1) % 3)
    wait_drain(0)
    wait_drain(1)
    wait_drain(2)


def _make_gather_body(nchunk):
    ew = nchunk * CH

    def body(u_hbm, v_hbm, dst_hbm, src_hbm, z_hbm,
             idx_d, idx_s, bu0, bv0, bu1, bv1, bu2, bv2,
             gs0, gs1, gs2, ws0, ws1, ws2):
        bus, bvs = (bu0, bu1, bu2), (bv0, bv1, bv2)
        gss, wss = (gs0, gs1, gs2), (ws0, ws1, ws2)
        c = lax.axis_index("c")
        s = lax.axis_index("s")
        wid = s * NC + c
        base0 = wid * ew
        cp_d = pltpu.async_copy(dst_hbm.at[pl.ds(base0, ew)], idx_d, gss[0])
        cp_s = pltpu.async_copy(src_hbm.at[pl.ds(base0, ew)], idx_s, gss[1])
        cp_d.wait()
        cp_s.wait()

        def prefetch(k, b):
            off = pl.ds(k * CH, CH)
            pltpu.async_copy(u_hbm.at[idx_d.at[off]], bus[b], gss[b])
            pltpu.async_copy(v_hbm.at[idx_s.at[off]], bvs[b], gss[b])

        def process(k, b):
            pltpu.make_async_copy(u_hbm.at[pl.ds(0, CH)], bus[b], gss[b]).wait()
            pltpu.make_async_copy(v_hbm.at[pl.ds(0, CH)], bvs[b], gss[b]).wait()

            def row(r, _):
                for j in range(D // 16):
                    sl = pl.ds(j * 16, 16)
                    plsc.addupdate(bus[b].at[r, sl], bvs[b][r, sl])
                return 0

            lax.fori_loop(0, CH, row, 0, unroll=4)
            pltpu.async_copy(bus[b], z_hbm.at[pl.ds(base0 + k * CH, CH)], wss[b])

        def wait_drain(b):
            pltpu.make_async_copy(z_hbm.at[pl.ds(0, CH)], bus[b], wss[b]).wait()

        _ring3(nchunk, process, prefetch, wait_drain)

    return body


def _run_gather(u, v, dst, src, nchunk):
    e_part = nchunk * CH * NW
    return pl.kernel(
        _make_gather_body(nchunk),
        mesh=_sc_mesh,
        out_type=jax.ShapeDtypeStruct((e_part, D), jnp.float32),
        scratch_types=[
            pltpu.VMEM((nchunk * CH,), jnp.int32),
            pltpu.VMEM((nchunk * CH,), jnp.int32),
            pltpu.VMEM((CH, D), jnp.float32),
            pltpu.VMEM((CH, D), jnp.float32),
            pltpu.VMEM((CH, D), jnp.float32),
            pltpu.VMEM((CH, D), jnp.float32),
            pltpu.VMEM((CH, D), jnp.float32),
            pltpu.VMEM((CH, D), jnp.float32),
            pltpu.SemaphoreType.DMA,
            pltpu.SemaphoreType.DMA,
            pltpu.SemaphoreType.DMA,
            pltpu.SemaphoreType.DMA,
            pltpu.SemaphoreType.DMA,
            pltpu.SemaphoreType.DMA,
        ],
    )(u, v, dst, src)


def _make_scatter_body(nchunk):
    ew = nchunk * CH

    def body(msg_hbm, dst3d_hbm, out_hbm, idx2d, m0, m1, m2, accum,
             isem, l0, l1, l2, s0, s1, s2):
        ms = (m0, m1, m2)
        lss, sss = (l0, l1, l2), (s0, s1, s2)
        c = lax.axis_index("c")
        s = lax.axis_index("s")
        wid = c * NS + s
        base0 = wid * ew

        cp_idx = pltpu.async_copy(dst3d_hbm.at[wid], idx2d, isem)

        def zrow(r, _):
            for j in range(D // 16):
                m0[r, pl.ds(j * 16, 16)] = jnp.zeros((16,), jnp.float32)
            return 0

        lax.fori_loop(0, CH, zrow, 0)
        for t in range(NROWS_W // CH):
            pltpu.sync_copy(m0, accum.at[pl.ds(s * NROWS_W + t * CH, CH)])
        cp_idx.wait()
        plsc.subcore_barrier()

        def prefetch(k, b):
            pltpu.async_copy(msg_hbm.at[pl.ds(base0 + k * CH, CH)], ms[b], lss[b])

        def process(k, b):
            pltpu.make_async_copy(msg_hbm.at[pl.ds(0, CH)], ms[b], lss[b]).wait()
            pltpu.async_copy(ms[b], accum.at[idx2d.at[k]], sss[b], add=True)

        def wait_drain(b):
            pltpu.make_async_copy(msg_hbm.at[pl.ds(0, CH)], ms[b], sss[b]).wait()

        _ring3(nchunk, process, prefetch, wait_drain)
        plsc.subcore_barrier()

        for t in range(NROWS_W // CH):
            sl = pl.ds(s * NROWS_W + t * CH, CH)
            pltpu.sync_copy(accum.at[sl], out_hbm.at[c, sl])

    return body


def _run_scatter(msg, dst3d, nchunk):
    return pl.kernel(
        _make_scatter_body(nchunk),
        mesh=_sc_mesh,
        out_type=jax.ShapeDtypeStruct((NC, NP, D), jnp.float32),
        scratch_types=[
            pltpu.VMEM((nchunk, CH), jnp.int32),
            pltpu.VMEM((CH, D), jnp.float32),
            pltpu.VMEM((CH, D), jnp.float32),
            pltpu.VMEM((CH, D), jnp.float32),
            pltpu.VMEM_SHARED((NP, D), jnp.float32),
            pltpu.SemaphoreType.DMA,
            pltpu.SemaphoreType.DMA,
            pltpu.SemaphoreType.DMA,
            pltpu.SemaphoreType.DMA,
            pltpu.SemaphoreType.DMA,
            pltpu.SemaphoreType.DMA,
            pltpu.SemaphoreType.DMA,
        ],
    )(msg, dst3d)


def kernel(pos, feat, edge_index, mW1, mb1, mg1, mbe1, mW2, mb2, mg2, mbe2,
           mW3, mb3, uW1, ub1, ug1, ube1, uW2, ub2, ug2, ube2, uW3, ub3):
    src = edge_index[0]
    dst = edge_index[1]
    x = jnp.concatenate([pos, feat], axis=1)

    s1 = mg1 / jnp.sqrt(1.0 + EPS)
    wp, wfi, wfj = mW1[0:2], mW1[2:D], mW1[D:2 * D - 2]
    wu = jnp.concatenate([-wp, wfi], axis=0) * s1
    wv = jnp.concatenate([wp, wfj], axis=0) * s1
    c1 = (mb1 * s1 + mbe1).reshape(1, D)

    s2 = mg2 / jnp.sqrt(1.0 + EPS)
    w2 = mW2 * s2
    b2 = (mb2 * s2 + mbe2).reshape(1, D)
    b3 = mb3.reshape(1, D)

    su1 = ug1 / jnp.sqrt(1.0 + EPS)
    w1a = uW1[:D] * su1
    w1b = uW1[D:] * su1
    cu1 = (ub1 * su1 + ube1).reshape(1, D)
    su2 = ug2 / jnp.sqrt(1.0 + EPS)
    uw2 = uW2 * su2
    bu2 = (ub2 * su2 + ube2).reshape(1, D)
    bu3 = ub3.reshape(1, D)

    u, v = _run_pre(x, wu, wv, c1)
    partial_list = []
    off = 0
    for units in PARTS:
        e_part = units * CH * NW
        dstp = lax.slice_in_dim(dst, off, off + e_part)
        srcp = lax.slice_in_dim(src, off, off + e_part)
        z = _run_gather(u, v, dstp, srcp, units)
        m = _run_edge_mlp(z, w2, b2, mW3, b3, e_part)
        partial_list.append(_run_scatter(m, dstp.reshape(NW, units, CH), units))
        off += e_part
    return _run_node_mlp(x, partial_list, w1a, w1b, cu1, uw2, bu2, uW3, bu3)


# speedup vs baseline: 1.0938x; 1.0444x over previous
import functools

import jax
import jax.numpy as jnp
from jax import lax
from jax.experimental import pallas as pl
from jax.experimental.pallas import tpu as pltpu
from jax.experimental.pallas import tpu_sc as plsc

N = 10000
E = 320000
D = 128
EPS = 1e-5

NC = 2
NS = 16
NW = NC * NS
EW = E // NW
CH = 80
NP = 10240
NROWS_W = NP // NS
ZCH = 128

BN_BLK = 1000
BE_BLK = 2560
PARTS = (42, 41, 42)

_sc_mesh = plsc.VectorSubcoreMesh(core_axis_name="c", subcore_axis_name="s")


def _pre_body(x_ref, wu_ref, wv_ref, c1_ref, u_ref, v_ref):
    x = x_ref[...]
    u_ref[...] = jnp.dot(x, wu_ref[...], preferred_element_type=jnp.float32) + c1_ref[...]
    v_ref[...] = jnp.dot(x, wv_ref[...], preferred_element_type=jnp.float32)


def _edge_body(z_ref, w2_ref, b2_ref, w3_ref, b3_ref, m_ref):
    h1 = jnp.maximum(z_ref[...], 0.0)
    h2 = jnp.dot(h1, w2_ref[...], preferred_element_type=jnp.float32) + b2_ref[...]
    h2 = jnp.maximum(h2, 0.0)
    m_ref[...] = jnp.dot(h2, w3_ref[...], preferred_element_type=jnp.float32) + b3_ref[...]


def _node_body(x_ref, *refs):
    nparts = len(PARTS)
    ps = refs[:nparts]
    w1a_ref, w1b_ref, c1_ref, w2_ref, b2_ref, w3_ref, b3_ref, o_ref = refs[nparts:]
    x = x_ref[...]
    a = ps[0][0] + ps[0][1]
    for p_ref in ps[1:]:
        a = a + p_ref[0] + p_ref[1]
    g = jnp.dot(x, w1a_ref[...], preferred_element_type=jnp.float32)
    g = g + jnp.dot(a, w1b_ref[...], preferred_element_type=jnp.float32)
    g = jnp.maximum(g + c1_ref[...], 0.0)
    g = jnp.dot(g, w2_ref[...], preferred_element_type=jnp.float32) + b2_ref[...]
    g = jnp.maximum(g, 0.0)
    o_ref[...] = jnp.dot(g, w3_ref[...], preferred_element_type=jnp.float32) + b3_ref[...]


_W_SPEC = pl.BlockSpec((D, D), lambda i: (0, 0))
_B_SPEC = pl.BlockSpec((1, D), lambda i: (0, 0))


BP_BLK = 2000


def _run_pre(x, wu, wv, c1):
    return pl.pallas_call(
        _pre_body,
        grid=(N // BP_BLK,),
        in_specs=[pl.BlockSpec((BP_BLK, D), lambda i: (i, 0)), _W_SPEC, _W_SPEC, _B_SPEC],
        out_specs=(pl.BlockSpec((BP_BLK, D), lambda i: (i, 0)),
                   pl.BlockSpec((BP_BLK, D), lambda i: (i, 0))),
        out_shape=(jax.ShapeDtypeStruct((N, D), jnp.float32),
                   jax.ShapeDtypeStruct((N, D), jnp.float32)),
    )(x, wu, wv, c1)


def _run_edge_mlp(z, w2, b2, w3, b3, e_part):
    return pl.pallas_call(
        _edge_body,
        grid=(e_part // BE_BLK,),
        in_specs=[pl.BlockSpec((BE_BLK, D), lambda i: (i, 0)), _W_SPEC, _B_SPEC, _W_SPEC, _B_SPEC],
        out_specs=pl.BlockSpec((BE_BLK, D), lambda i: (i, 0)),
        out_shape=jax.ShapeDtypeStruct((e_part, D), jnp.float32),
    )(z, w2, b2, w3, b3)


def _run_node_mlp(x, partial_list, w1a, w1b, c1, w2, b2, w3, b3):
    p_spec = pl.BlockSpec((NC, BN_BLK, D), lambda i: (0, i, 0))
    return pl.pallas_call(
        _node_body,
        grid=(N // BN_BLK,),
        in_specs=[pl.BlockSpec((BN_BLK, D), lambda i: (i, 0))]
                 + [p_spec] * len(partial_list)
                 + [_W_SPEC, _W_SPEC, _B_SPEC, _W_SPEC, _B_SPEC, _W_SPEC, _B_SPEC],
        out_specs=pl.BlockSpec((BN_BLK, D), lambda i: (i, 0)),
        out_shape=jax.ShapeDtypeStruct((N, D), jnp.float32),
    )(x, *partial_list, w1a, w1b, c1, w2, b2, w3, b3)


NCHUNK = EW // CH


def _ring3(nchunk, process, prefetch, wait_drain):
    prefetch(0, 0)
    prefetch(1, 1)
    process(0, 0)
    prefetch(2, 2)
    process(1, 1)
    wait_drain(0)
    prefetch(3, 0)

    def full(k, b):
        process(k, b)
        wait_drain((b + 2) % 3)
        prefetch(k + 2, (b + 2) % 3)

    body_count = nchunk - 4
    triples = body_count // 3
    rem = body_count % 3

    if triples > 0:
        def triple(g, _):
            k0 = 2 + 3 * g
            full(k0, 2)
            full(k0 + 1, 0)
            full(k0 + 2, 1)
            return 0

        lax.fori_loop(0, triples, triple, 0)
    for j in range(rem):
        k = 2 + 3 * triples + j
        full(k, k % 3)
    process(nchunk - 2, (nchunk - 2) % 3)
    process(nchunk - 1, (nchunk - 1) % 3)
    wait_drain(0)
    wait_drain(1)
    wait_drain(2)


def _make_gather_body(nchunk):
    ew = nchunk * CH

    def body(u_hbm, v_hbm, dst_hbm, src_hbm, z_hbm,
             idx_d, idx_s, bu0, bv0, bu1, bv1, bu2, bv2,
             gs0, gs1, gs2, ws0, ws1, ws2):
        bus, bvs = (bu0, bu1, bu2), (bv0, bv1, bv2)
        gss, wss = (gs0, gs1, gs2), (ws0, ws1, ws2)
        c = lax.axis_index("c")
        s = lax.axis_index("s")
        wid = s * NC + c
        base0 = wid * ew
        cp_d = pltpu.async_copy(dst_hbm.at[pl.ds(base0, ew)], idx_d, gss[0])
        cp_s = pltpu.async_copy(src_hbm.at[pl.ds(base0, ew)], idx_s, gss[1])
        cp_d.wait()
        cp_s.wait()

        def prefetch(k, b):
            off = pl.ds(k * CH, CH)
            pltpu.async_copy(u_hbm.at[idx_d.at[off]], bus[b], gss[b])
            pltpu.async_copy(v_hbm.at[idx_s.at[off]], bvs[b], gss[b])

        def process(k, b):
            pltpu.make_async_copy(u_hbm.at[pl.ds(0, CH)], bus[b], gss[b]).wait()
            pltpu.make_async_copy(v_hbm.at[pl.ds(0, CH)], bvs[b], gss[b]).wait()

            def row(r, _):
                for j in range(D // 16):
                    sl = pl.ds(j * 16, 16)
                    plsc.addupdate(bus[b].at[r, sl], bvs[b][r, sl])
                return 0

            lax.fori_loop(0, CH, row, 0, unroll=4)
            pltpu.async_copy(bus[b], z_hbm.at[pl.ds(base0 + k * CH, CH)], wss[b])

        def wait_drain(b):
            pltpu.make_async_copy(z_hbm.at[pl.ds(0, CH)], bus[b], wss[b]).wait()

        _ring3(nchunk, process, prefetch, wait_drain)

    return body


def _run_gather(u, v, dst, src, nchunk):
    e_part = nchunk * CH * NW
    return pl.kernel(
        _make_gather_body(nchunk),
        mesh=_sc_mesh,
        out_type=jax.ShapeDtypeStruct((e_part, D), jnp.float32),
        scratch_types=[
            pltpu.VMEM((nchunk * CH,), jnp.int32),
            pltpu.VMEM((nchunk * CH,), jnp.int32),
            pltpu.VMEM((CH, D), jnp.float32),
            pltpu.VMEM((CH, D), jnp.float32),
            pltpu.VMEM((CH, D), jnp.float32),
            pltpu.VMEM((CH, D), jnp.float32),
            pltpu.VMEM((CH, D), jnp.float32),
            pltpu.VMEM((CH, D), jnp.float32),
            pltpu.SemaphoreType.DMA,
            pltpu.SemaphoreType.DMA,
            pltpu.SemaphoreType.DMA,
            pltpu.SemaphoreType.DMA,
            pltpu.SemaphoreType.DMA,
            pltpu.SemaphoreType.DMA,
        ],
    )(u, v, dst, src)


def _make_scatter_body(nchunk):
    ew = nchunk * CH

    def body(msg_hbm, dst3d_hbm, out_hbm, idx2d, m0, m1, m2, accum,
             isem, l0, l1, l2, s0, s1, s2):
        ms = (m0, m1, m2)
        lss, sss = (l0, l1, l2), (s0, s1, s2)
        c = lax.axis_index("c")
        s = lax.axis_index("s")
        wid = c * NS + s
        base0 = wid * ew

        cp_idx = pltpu.async_copy(dst3d_hbm.at[wid], idx2d, isem)

        def zrow(r, _):
            for j in range(D // 16):
                m0[r, pl.ds(j * 16, 16)] = jnp.zeros((16,), jnp.float32)
            return 0

        lax.fori_loop(0, CH, zrow, 0)
        for t in range(NROWS_W // CH):
            pltpu.sync_copy(m0, accum.at[pl.ds(s * NROWS_W + t * CH, CH)])
        cp_idx.wait()
        plsc.subcore_barrier()

        def prefetch(k, b):
            pltpu.async_copy(msg_hbm.at[pl.ds(base0 + k * CH, CH)], ms[b], lss[b])

        def process(k, b):
            pltpu.make_async_copy(msg_hbm.at[pl.ds(0, CH)], ms[b], lss[b]).wait()
            pltpu.async_copy(ms[b], accum.at[idx2d.at[k]], sss[b], add=True)

        def wait_drain(b):
            pltpu.make_async_copy(msg_hbm.at[pl.ds(0, CH)], ms[b], sss[b]).wait()

        _ring3(nchunk, process, prefetch, wait_drain)
        plsc.subcore_barrier()

        for t in range(NROWS_W // CH):
            sl = pl.ds(s * NROWS_W + t * CH, CH)
            pltpu.sync_copy(accum.at[sl], out_hbm.at[c, sl])

    return body


def _run_scatter(msg, dst3d, nchunk):
    return pl.kernel(
        _make_scatter_body(nchunk),
        mesh=_sc_mesh,
        out_type=jax.ShapeDtypeStruct((NC, NP, D), jnp.float32),
        scratch_types=[
            pltpu.VMEM((nchunk, CH), jnp.int32),
            pltpu.VMEM((CH, D), jnp.float32),
            pltpu.VMEM((CH, D), jnp.float32),
            pltpu.VMEM((CH, D), jnp.float32),
            pltpu.VMEM_SHARED((NP, D), jnp.float32),
            pltpu.SemaphoreType.DMA,
            pltpu.SemaphoreType.DMA,
            pltpu.SemaphoreType.DMA,
            pltpu.SemaphoreType.DMA,
            pltpu.SemaphoreType.DMA,
            pltpu.SemaphoreType.DMA,
            pltpu.SemaphoreType.DMA,
        ],
    )(msg, dst3d)


def kernel(pos, feat, edge_index, mW1, mb1, mg1, mbe1, mW2, mb2, mg2, mbe2,
           mW3, mb3, uW1, ub1, ug1, ube1, uW2, ub2, ug2, ube2, uW3, ub3):
    src = edge_index[0]
    dst = edge_index[1]
    x = jnp.concatenate([pos, feat], axis=1)

    s1 = mg1 / jnp.sqrt(1.0 + EPS)
    wp, wfi, wfj = mW1[0:2], mW1[2:D], mW1[D:2 * D - 2]
    wu = jnp.concatenate([-wp, wfi], axis=0) * s1
    wv = jnp.concatenate([wp, wfj], axis=0) * s1
    c1 = (mb1 * s1 + mbe1).reshape(1, D)

    s2 = mg2 / jnp.sqrt(1.0 + EPS)
    w2 = mW2 * s2
    b2 = (mb2 * s2 + mbe2).reshape(1, D)
    b3 = mb3.reshape(1, D)

    su1 = ug1 / jnp.sqrt(1.0 + EPS)
    w1a = uW1[:D] * su1
    w1b = uW1[D:] * su1
    cu1 = (ub1 * su1 + ube1).reshape(1, D)
    su2 = ug2 / jnp.sqrt(1.0 + EPS)
    uw2 = uW2 * su2
    bu2 = (ub2 * su2 + ube2).reshape(1, D)
    bu3 = ub3.reshape(1, D)

    u, v = _run_pre(x, wu, wv, c1)
    partial_list = []
    off = 0
    for units in PARTS:
        e_part = units * CH * NW
        dstp = lax.slice_in_dim(dst, off, off + e_part)
        srcp = lax.slice_in_dim(src, off, off + e_part)
        z = _run_gather(u, v, dstp, srcp, units)
        m = _run_edge_mlp(z, w2, b2, mW3, b3, e_part)
        partial_list.append(_run_scatter(m, dstp.reshape(NW, units, CH), units))
        off += e_part
    return _run_node_mlp(x, partial_list, w1a, w1b, cu1, uw2, bu2, uW3, bu3)
